# baseline (speedup 1.0000x reference)
import functools

import jax
import jax.numpy as jnp
from jax import lax
from jax.experimental import pallas as pl
from jax.experimental.pallas import tpu as pltpu
from jax.experimental.pallas import tpu_sc as plsc

B, L, G = 1024, 200, 3
D = 64
N = B * L * G
LANES = 16
NW = 32
ROWS_PER_W = N // NW
C = 128
N_CHUNKS = ROWS_PER_W // C
NBUF = 6
K = 3


def _relu_body(t_ref, o_ref):
    o_ref[...] = jnp.maximum(t_ref[...], 0.0)


def _relu_table(table):
    V = table.shape[0]
    BLK = 8192
    return pl.pallas_call(
        _relu_body,
        grid=(pl.cdiv(V, BLK),),
        in_specs=[pl.BlockSpec((BLK, D), lambda i: (i, 0))],
        out_specs=pl.BlockSpec((BLK, D), lambda i: (i, 0)),
        out_shape=jax.ShapeDtypeStruct((V, D), table.dtype),
    )(table)


_mesh = plsc.VectorSubcoreMesh(core_axis_name="c", subcore_axis_name="s")

_scratch = (
    [pltpu.VMEM((C,), jnp.int32) for _ in range(NBUF)]
    + [pltpu.VMEM((C, D), jnp.float32) for _ in range(NBUF)]
    + [pltpu.SemaphoreType.DMA for _ in range(3 * NBUF)]
)


@functools.partial(
    pl.kernel,
    out_type=jax.ShapeDtypeStruct((N, D), jnp.float32),
    mesh=_mesh,
    scratch_types=_scratch,
    compiler_params=pltpu.CompilerParams(use_tc_tiling_on_sc=False),
)
def _gather_scatter(idx_hbm, table_hbm, out_hbm, *scratch):
    ibuf = scratch[0:NBUF]
    rows = scratch[NBUF : 2 * NBUF]
    gsem = scratch[2 * NBUF : 3 * NBUF]
    ssem = scratch[3 * NBUF : 4 * NBUF]
    isem = scratch[4 * NBUF : 5 * NBUF]

    cid = lax.axis_index("c")
    sid = lax.axis_index("s")
    wid = sid * 2 + cid
    wbase = wid * ROWS_PER_W
    wchunk = wid * N_CHUNKS

    def idx_start(c, b):
        base = pl.multiple_of(wbase + c * C, C)
        pltpu.async_copy(idx_hbm.at[pl.ds(base, C)], ibuf[b], isem[b])

    def idx_wait(b):
        pltpu.make_async_copy(idx_hbm.at[pl.ds(0, C)], ibuf[b], isem[b]).wait()

    def gather_start(b):
        pltpu.async_copy(table_hbm.at[ibuf[b]], rows[b], gsem[b])

    def gather_wait(b):
        pltpu.make_async_copy(table_hbm.at[ibuf[b]], rows[b], gsem[b]).wait()

    def scatter_start(c, b):
        base = pl.multiple_of((wchunk + c) * C, C)
        pltpu.async_copy(rows[b], out_hbm.at[pl.ds(base, C)], ssem[b])

    def scatter_wait(b):
        pltpu.make_async_copy(rows[b], out_hbm.at[pl.ds(0, C)], ssem[b]).wait()

    for c in range(NBUF):
        idx_start(c, c)
    for c in range(NBUF):
        b = c
        if c >= K:
            b2 = c - K
            gather_wait(b2)
            scatter_start(c - K, b2)
            idx_start(c + K, b2)
        idx_wait(b)
        gather_start(b)

    @pl.loop(0, (N_CHUNKS - 2 * NBUF) // NBUF)
    def _block(j):
        for b in range(NBUF):
            c = NBUF + j * NBUF + b
            b2 = (b + NBUF - K) % NBUF
            gather_wait(b2)
            scatter_start(c - K, b2)
            idx_start(c + K, b2)
            scatter_wait(b)
            idx_wait(b)
            gather_start(b)

    for c in range(N_CHUNKS - NBUF, N_CHUNKS):
        b = c % NBUF
        b2 = (b + NBUF - K) % NBUF
        gather_wait(b2)
        scatter_start(c - K, b2)
        if c + K < N_CHUNKS:
            idx_start(c + K, b2)
        scatter_wait(b)
        idx_wait(b)
        gather_start(b)

    for c in range(N_CHUNKS - K, N_CHUNKS):
        b = c % NBUF
        gather_wait(b)
        scatter_start(c, b)
    for b in range(NBUF):
        scatter_wait(b)


def kernel(indices, table):
    rtable = _relu_table(table)
    idx_flat = jnp.transpose(indices.astype(jnp.int32), (2, 1, 0)).reshape(-1)
    out = _gather_scatter(idx_flat, rtable)
    return jnp.transpose(out.reshape(G, L, B, D), (1, 2, 0, 3)).reshape(L, B, G * D)


# speedup vs baseline: 1.0794x; 1.0794x over previous
import functools

import jax
import jax.numpy as jnp
from jax import lax
from jax.experimental import pallas as pl
from jax.experimental.pallas import tpu as pltpu
from jax.experimental.pallas import tpu_sc as plsc

B, L, G = 1024, 200, 3
D = 64
N = B * L * G
LANES = 16
NW = 32
ROWS_PER_W = N // NW
C = 128
N_CHUNKS = ROWS_PER_W // C
NBUF = 6
K = 3


def _relu_body(t_ref, o_ref):
    o_ref[...] = jnp.maximum(t_ref[...], 0.0)


def _relu_table(table):
    V = table.shape[0]
    BLK = 8192
    return pl.pallas_call(
        _relu_body,
        grid=(pl.cdiv(V, BLK),),
        in_specs=[pl.BlockSpec((BLK, D), lambda i: (i, 0))],
        out_specs=pl.BlockSpec((BLK, D), lambda i: (i, 0)),
        out_shape=jax.ShapeDtypeStruct((V, D), table.dtype),
    )(table)


_mesh = plsc.VectorSubcoreMesh(core_axis_name="c", subcore_axis_name="s")

_scratch = (
    [pltpu.VMEM((C,), jnp.int32) for _ in range(NBUF)]
    + [pltpu.VMEM((C,), jnp.int32) for _ in range(NBUF)]
    + [pltpu.VMEM((C, D), jnp.float32) for _ in range(NBUF)]
    + [pltpu.SemaphoreType.DMA for _ in range(3 * NBUF)]
)


@functools.partial(
    pl.kernel,
    out_type=jax.ShapeDtypeStruct((N, D), jnp.float32),
    mesh=_mesh,
    scratch_types=_scratch,
    compiler_params=pltpu.CompilerParams(use_tc_tiling_on_sc=False),
)
def _gather_scatter(idx_hbm, table_hbm, out_hbm, *scratch):
    ibuf = scratch[0:NBUF]
    sidx = scratch[NBUF : 2 * NBUF]
    rows = scratch[2 * NBUF : 3 * NBUF]
    gsem = scratch[3 * NBUF : 4 * NBUF]
    ssem = scratch[4 * NBUF : 5 * NBUF]
    isem = scratch[5 * NBUF : 6 * NBUF]

    cid = lax.axis_index("c")
    sid = lax.axis_index("s")
    wid = sid * 2 + cid
    wbase = wid * ROWS_PER_W

    def compute_sidx(b, cg):
        c8 = lax.shift_right_logical(cg, 3)
        g = lax.shift_right_logical(c8 * 328, 16)
        l = c8 - L * g
        b0 = lax.shift_left(cg & 7, 7)
        base_s = l * (B * G) + b0 * G + g
        for v in range(C // LANES):
            cv = (lax.iota(jnp.int32, LANES) + (v * LANES)) * G
            sidx[b][pl.ds(v * LANES, LANES)] = base_s + cv

    def idx_start(c, b):
        base = pl.multiple_of(wbase + c * C, C)
        pltpu.async_copy(idx_hbm.at[pl.ds(base, C)], ibuf[b], isem[b])

    def idx_wait(b):
        pltpu.make_async_copy(idx_hbm.at[pl.ds(0, C)], ibuf[b], isem[b]).wait()

    def gather_start(b):
        pltpu.async_copy(table_hbm.at[ibuf[b]], rows[b], gsem[b])

    def gather_wait(b):
        pltpu.make_async_copy(table_hbm.at[ibuf[b]], rows[b], gsem[b]).wait()

    def scatter_start(b):
        pltpu.async_copy(rows[b], out_hbm.at[sidx[b]], ssem[b])

    def scatter_wait(b):
        pltpu.make_async_copy(rows[b], out_hbm.at[sidx[b]], ssem[b]).wait()

    wchunk = wid * N_CHUNKS

    for c in range(NBUF):
        idx_start(c, c)
    for c in range(NBUF):
        b = c
        if c >= K:
            b2 = c - K
            gather_wait(b2)
            scatter_start(b2)
            idx_start(c + K, b2)
        compute_sidx(b, wchunk + c)
        idx_wait(b)
        gather_start(b)

    @pl.loop(0, (N_CHUNKS - 2 * NBUF) // NBUF)
    def _block(j):
        for b in range(NBUF):
            c = NBUF + j * NBUF + b
            b2 = (b + NBUF - K) % NBUF
            gather_wait(b2)
            scatter_start(b2)
            idx_start(c + K, b2)
            scatter_wait(b)
            compute_sidx(b, wchunk + c)
            idx_wait(b)
            gather_start(b)

    for c in range(N_CHUNKS - NBUF, N_CHUNKS):
        b = c % NBUF
        b2 = (b + NBUF - K) % NBUF
        gather_wait(b2)
        scatter_start(b2)
        if c + K < N_CHUNKS:
            idx_start(c + K, b2)
        scatter_wait(b)
        compute_sidx(b, wchunk + c)
        idx_wait(b)
        gather_start(b)

    for c in range(N_CHUNKS - K, N_CHUNKS):
        b = c % NBUF
        gather_wait(b)
        scatter_start(b)
    for b in range(NBUF):
        scatter_wait(b)


def kernel(indices, table):
    rtable = _relu_table(table)
    idx_flat = jnp.transpose(indices.astype(jnp.int32), (2, 1, 0)).reshape(-1)
    out = _gather_scatter(idx_flat, rtable)
    return out.reshape(L, B, G * D)


# speedup vs baseline: 1.4061x; 1.3027x over previous
import functools

import jax
import jax.numpy as jnp
from jax import lax
from jax.experimental import pallas as pl
from jax.experimental.pallas import tpu as pltpu
from jax.experimental.pallas import tpu_sc as plsc

B, L, G = 1024, 200, 3
D = 64
N = B * L * G
LANES = 16
NW = 32
ROWS_PER_W = N // NW
C = 128
N_CHUNKS = ROWS_PER_W // C
NBUF = 6
K = 3


def _relu_body(t_ref, o_ref):
    o_ref[...] = jnp.maximum(t_ref[...], 0.0)


def _relu_table(table):
    V = table.shape[0]
    BLK = 8192
    return pl.pallas_call(
        _relu_body,
        grid=(pl.cdiv(V, BLK),),
        in_specs=[pl.BlockSpec((BLK, D), lambda i: (i, 0))],
        out_specs=pl.BlockSpec((BLK, D), lambda i: (i, 0)),
        out_shape=jax.ShapeDtypeStruct((V, D), table.dtype),
    )(table)


_mesh = plsc.VectorSubcoreMesh(core_axis_name="c", subcore_axis_name="s")

_scratch = (
    [pltpu.VMEM((C,), jnp.int32) for _ in range(NBUF)]
    + [pltpu.VMEM((C,), jnp.int32) for _ in range(NBUF)]
    + [pltpu.VMEM((C, D), jnp.float32) for _ in range(NBUF)]
    + [pltpu.SemaphoreType.DMA for _ in range(3 * NBUF)]
)


@functools.partial(
    pl.kernel,
    out_type=jax.ShapeDtypeStruct((N, D), jnp.float32),
    mesh=_mesh,
    scratch_types=_scratch,
    compiler_params=pltpu.CompilerParams(use_tc_tiling_on_sc=False),
)
def _gather_scatter(idx_hbm, table_hbm, out_hbm, *scratch):
    ibuf = scratch[0:NBUF]
    sidx = scratch[NBUF : 2 * NBUF]
    rows = scratch[2 * NBUF : 3 * NBUF]
    gsem = scratch[3 * NBUF : 4 * NBUF]
    ssem = scratch[4 * NBUF : 5 * NBUF]
    isem = scratch[5 * NBUF : 6 * NBUF]

    cid = lax.axis_index("c")
    sid = lax.axis_index("s")
    wid = sid * 2 + cid
    wbase = wid * ROWS_PER_W

    def compute_sidx(b, cg):
        c8 = lax.shift_right_logical(cg, 3)
        g = lax.shift_right_logical(c8 * 328, 16)
        l = c8 - L * g
        b0 = lax.shift_left(cg & 7, 7)
        base_s = (
            l * (B * G)
            + g * B
            + lax.shift_left(b0 & 511, 1)
            + lax.shift_right_logical(b0, 9)
        )
        for v in range(C // LANES):
            cv = (lax.iota(jnp.int32, LANES) + (v * LANES)) * 2
            sidx[b][pl.ds(v * LANES, LANES)] = base_s + cv

    def idx_start(c, b):
        base = pl.multiple_of(wbase + c * C, C)
        pltpu.async_copy(idx_hbm.at[pl.ds(base, C)], ibuf[b], isem[b])

    def idx_wait(b):
        pltpu.make_async_copy(idx_hbm.at[pl.ds(0, C)], ibuf[b], isem[b]).wait()

    def gather_start(b):
        pltpu.async_copy(table_hbm.at[ibuf[b]], rows[b], gsem[b])

    def gather_wait(b):
        pltpu.make_async_copy(table_hbm.at[ibuf[b]], rows[b], gsem[b]).wait()

    def scatter_start(b):
        pltpu.async_copy(rows[b], out_hbm.at[sidx[b]], ssem[b])

    def scatter_wait(b):
        pltpu.make_async_copy(rows[b], out_hbm.at[sidx[b]], ssem[b]).wait()

    wchunk = wid * N_CHUNKS

    for c in range(NBUF):
        idx_start(c, c)
    for c in range(NBUF):
        b = c
        if c >= K:
            b2 = c - K
            gather_wait(b2)
            scatter_start(b2)
            idx_start(c + K, b2)
        compute_sidx(b, wchunk + c)
        idx_wait(b)
        gather_start(b)

    @pl.loop(0, (N_CHUNKS - 2 * NBUF) // NBUF)
    def _block(j):
        for b in range(NBUF):
            c = NBUF + j * NBUF + b
            b2 = (b + NBUF - K) % NBUF
            gather_wait(b2)
            scatter_start(b2)
            idx_start(c + K, b2)
            scatter_wait(b)
            compute_sidx(b, wchunk + c)
            idx_wait(b)
            gather_start(b)

    for c in range(N_CHUNKS - NBUF, N_CHUNKS):
        b = c % NBUF
        b2 = (b + NBUF - K) % NBUF
        gather_wait(b2)
        scatter_start(b2)
        if c + K < N_CHUNKS:
            idx_start(c + K, b2)
        scatter_wait(b)
        compute_sidx(b, wchunk + c)
        idx_wait(b)
        gather_start(b)

    for c in range(N_CHUNKS - K, N_CHUNKS):
        b = c % NBUF
        gather_wait(b)
        scatter_start(b)
    for b in range(NBUF):
        scatter_wait(b)


def _finish_body(x_ref, o_ref):
    for g in range(G):
        xg = x_ref[0, g * 512 : (g + 1) * 512, :]
        xt = xg.T
        o_ref[0, g * D : (g + 1) * D, :] = jnp.concatenate(
            [xt[:D, :], xt[D:, :]], axis=1
        )


def _finisher(out_lin):
    x = out_lin.reshape(L, (B * G * D) // 128, 128)
    z = pl.pallas_call(
        _finish_body,
        grid=(L,),
        in_specs=[pl.BlockSpec((1, (B * G * D) // 128, 128), lambda i: (i, 0, 0))],
        out_specs=pl.BlockSpec((1, G * D, B), lambda i: (i, 0, 0)),
        out_shape=jax.ShapeDtypeStruct((L, G * D, B), jnp.float32),
    )(x)
    return jnp.transpose(z, (0, 2, 1))


def kernel(indices, table):
    rtable = _relu_table(table)
    idx_flat = jnp.transpose(indices.astype(jnp.int32), (2, 1, 0)).reshape(-1)
    out = _gather_scatter(idx_flat, rtable)
    return _finisher(out)


# speedup vs baseline: 1.8474x; 1.3138x over previous
import functools

import jax
import jax.numpy as jnp
from jax import lax
from jax.experimental import pallas as pl
from jax.experimental.pallas import tpu as pltpu
from jax.experimental.pallas import tpu_sc as plsc

B, L, G = 1024, 200, 3
D = 64
N = B * L * G
LANES = 16
NW = 32
ROWS_PER_W = N // NW
C = 128
N_CHUNKS = ROWS_PER_W // C
NBUF = 6
K = 3


_mesh = plsc.VectorSubcoreMesh(core_axis_name="c", subcore_axis_name="s")

_scratch = (
    [pltpu.VMEM((C,), jnp.int32) for _ in range(NBUF)]
    + [pltpu.VMEM((C,), jnp.int32) for _ in range(NBUF)]
    + [pltpu.VMEM((C, D), jnp.float32) for _ in range(NBUF)]
    + [pltpu.SemaphoreType.DMA for _ in range(3 * NBUF)]
)


@functools.partial(
    pl.kernel,
    out_type=jax.ShapeDtypeStruct((N, D), jnp.float32),
    mesh=_mesh,
    scratch_types=_scratch,
    compiler_params=pltpu.CompilerParams(use_tc_tiling_on_sc=False),
)
def _gather_scatter(idx_hbm, table_hbm, out_hbm, *scratch):
    ibuf = scratch[0:NBUF]
    sidx = scratch[NBUF : 2 * NBUF]
    rows = scratch[2 * NBUF : 3 * NBUF]
    gsem = scratch[3 * NBUF : 4 * NBUF]
    ssem = scratch[4 * NBUF : 5 * NBUF]
    isem = scratch[5 * NBUF : 6 * NBUF]

    cid = lax.axis_index("c")
    sid = lax.axis_index("s")
    wid = sid * 2 + cid
    wbase = wid * ROWS_PER_W

    def compute_sidx(b, cg):
        c8 = lax.shift_right_logical(cg, 3)
        g = lax.shift_right_logical(c8 * 328, 16)
        l = c8 - L * g
        b0 = lax.shift_left(cg & 7, 7)
        base_s = (
            l * (B * G)
            + g * B
            + lax.shift_left(b0 & 511, 1)
            + lax.shift_right_logical(b0, 9)
        )
        for v in range(C // LANES):
            cv = (lax.iota(jnp.int32, LANES) + (v * LANES)) * 2
            sidx[b][pl.ds(v * LANES, LANES)] = base_s + cv

    def idx_start(c, b):
        base = pl.multiple_of(wbase + c * C, C)
        pltpu.async_copy(idx_hbm.at[pl.ds(base, C)], ibuf[b], isem[b])

    def idx_wait(b):
        pltpu.make_async_copy(idx_hbm.at[pl.ds(0, C)], ibuf[b], isem[b]).wait()

    def gather_start(b):
        pltpu.async_copy(table_hbm.at[ibuf[b]], rows[b], gsem[b])

    def gather_wait(b):
        pltpu.make_async_copy(table_hbm.at[ibuf[b]], rows[b], gsem[b]).wait()

    def scatter_start(b):
        pltpu.async_copy(rows[b], out_hbm.at[sidx[b]], ssem[b])

    def scatter_wait(b):
        pltpu.make_async_copy(rows[b], out_hbm.at[sidx[b]], ssem[b]).wait()

    wchunk = wid * N_CHUNKS

    for c in range(NBUF):
        idx_start(c, c)
    for c in range(NBUF):
        b = c
        if c >= K:
            b2 = c - K
            gather_wait(b2)
            scatter_start(b2)
            idx_start(c + K, b2)
        compute_sidx(b, wchunk + c)
        idx_wait(b)
        gather_start(b)

    @pl.loop(0, (N_CHUNKS - 2 * NBUF) // NBUF)
    def _block(j):
        for b in range(NBUF):
            c = NBUF + j * NBUF + b
            b2 = (b + NBUF - K) % NBUF
            gather_wait(b2)
            scatter_start(b2)
            idx_start(c + K, b2)
            scatter_wait(b)
            compute_sidx(b, wchunk + c)
            idx_wait(b)
            gather_start(b)

    for c in range(N_CHUNKS - NBUF, N_CHUNKS):
        b = c % NBUF
        b2 = (b + NBUF - K) % NBUF
        gather_wait(b2)
        scatter_start(b2)
        if c + K < N_CHUNKS:
            idx_start(c + K, b2)
        scatter_wait(b)
        compute_sidx(b, wchunk + c)
        idx_wait(b)
        gather_start(b)

    for c in range(N_CHUNKS - K, N_CHUNKS):
        b = c % NBUF
        gather_wait(b)
        scatter_start(b)
    for b in range(NBUF):
        scatter_wait(b)


_LBLK = 2


def _finish_body(x_ref, o_ref):
    for l in range(_LBLK):
        for g in range(G):
            xg = x_ref[l, g * 512 : (g + 1) * 512, :]
            xt = xg.T
            o_ref[l, g * D : (g + 1) * D, :] = jnp.maximum(
                jnp.concatenate([xt[:D, :], xt[D:, :]], axis=1), 0.0
            )


def _finisher(out_lin):
    x = out_lin.reshape(L, (B * G * D) // 128, 128)
    z = pl.pallas_call(
        _finish_body,
        grid=(L // _LBLK,),
        in_specs=[
            pl.BlockSpec((_LBLK, (B * G * D) // 128, 128), lambda i: (i, 0, 0))
        ],
        out_specs=pl.BlockSpec((_LBLK, G * D, B), lambda i: (i, 0, 0)),
        out_shape=jax.ShapeDtypeStruct((L, G * D, B), jnp.float32),
    )(x)
    return jnp.transpose(z, (0, 2, 1))


def kernel(indices, table):
    idx_flat = jnp.transpose(indices.astype(jnp.int32), (2, 1, 0)).reshape(-1)
    out = _gather_scatter(idx_flat, table.astype(jnp.float32))
    return _finisher(out)


# speedup vs baseline: 2.0222x; 1.0946x over previous
import functools

import jax
import jax.numpy as jnp
from jax import lax
from jax.experimental import pallas as pl
from jax.experimental.pallas import tpu as pltpu
from jax.experimental.pallas import tpu_sc as plsc

B, L, G = 1024, 200, 3
D = 64
N = B * L * G
LANES = 16
NW = 32
ROWS_PER_W = N // NW
C = 128
N_CHUNKS = ROWS_PER_W // C
NBUF = 6
K = 3


_mesh = plsc.VectorSubcoreMesh(core_axis_name="c", subcore_axis_name="s")

_scratch = (
    [pltpu.VMEM((C,), jnp.int32) for _ in range(NBUF)]
    + [pltpu.VMEM((C,), jnp.int32) for _ in range(NBUF)]
    + [pltpu.VMEM((C, D), jnp.float32) for _ in range(NBUF)]
    + [pltpu.SemaphoreType.DMA for _ in range(3 * NBUF)]
)


@functools.partial(
    pl.kernel,
    out_type=jax.ShapeDtypeStruct((N, D), jnp.float32),
    mesh=_mesh,
    scratch_types=_scratch,
    compiler_params=pltpu.CompilerParams(use_tc_tiling_on_sc=False),
)
def _gather_scatter(idx_hbm, table_hbm, out_hbm, *scratch):
    ibuf = scratch[0:NBUF]
    sidx = scratch[NBUF : 2 * NBUF]
    rows = scratch[2 * NBUF : 3 * NBUF]
    gsem = scratch[3 * NBUF : 4 * NBUF]
    ssem = scratch[4 * NBUF : 5 * NBUF]
    isem = scratch[5 * NBUF : 6 * NBUF]

    cid = lax.axis_index("c")
    sid = lax.axis_index("s")
    wid = sid * 2 + cid
    wbase = wid * ROWS_PER_W

    def compute_sidx(b, cg):
        c8 = lax.shift_right_logical(cg, 3)
        g = lax.shift_right_logical(c8 * 328, 16)
        l = c8 - L * g
        b0 = lax.shift_left(cg & 7, 7)
        base_s = (
            l * (B * G)
            + g * B
            + lax.shift_left(b0 & 511, 1)
            + lax.shift_right_logical(b0, 9)
        )
        for v in range(C // LANES):
            cv = (lax.iota(jnp.int32, LANES) + (v * LANES)) * 2
            sidx[b][pl.ds(v * LANES, LANES)] = base_s + cv

    def idx_start(c, b):
        base = pl.multiple_of(wbase + c * C, C)
        pltpu.async_copy(idx_hbm.at[pl.ds(base, C)], ibuf[b], isem[b])

    def idx_wait(b):
        pltpu.make_async_copy(idx_hbm.at[pl.ds(0, C)], ibuf[b], isem[b]).wait()

    def gather_start(b):
        pltpu.async_copy(table_hbm.at[ibuf[b]], rows[b], gsem[b])

    def gather_wait(b):
        pltpu.make_async_copy(table_hbm.at[ibuf[b]], rows[b], gsem[b]).wait()

    def scatter_start(b):
        pltpu.async_copy(rows[b], out_hbm.at[sidx[b]], ssem[b])

    def scatter_wait(b):
        pltpu.make_async_copy(rows[b], out_hbm.at[sidx[b]], ssem[b]).wait()

    wchunk = wid * N_CHUNKS

    for c in range(NBUF):
        idx_start(c, c)
    for c in range(NBUF):
        b = c
        if c >= K:
            b2 = c - K
            gather_wait(b2)
            scatter_start(b2)
            idx_start(c + K, b2)
        compute_sidx(b, wchunk + c)
        idx_wait(b)
        gather_start(b)

    @pl.loop(0, (N_CHUNKS - 2 * NBUF) // NBUF)
    def _block(j):
        for b in range(NBUF):
            c = NBUF + j * NBUF + b
            b2 = (b + NBUF - K) % NBUF
            gather_wait(b2)
            scatter_start(b2)
            idx_start(c + K, b2)
            scatter_wait(b)
            compute_sidx(b, wchunk + c)
            idx_wait(b)
            gather_start(b)

    for c in range(N_CHUNKS - NBUF, N_CHUNKS):
        b = c % NBUF
        b2 = (b + NBUF - K) % NBUF
        gather_wait(b2)
        scatter_start(b2)
        if c + K < N_CHUNKS:
            idx_start(c + K, b2)
        scatter_wait(b)
        compute_sidx(b, wchunk + c)
        idx_wait(b)
        gather_start(b)

    for c in range(N_CHUNKS - K, N_CHUNKS):
        b = c % NBUF
        gather_wait(b)
        scatter_start(b)
    for b in range(NBUF):
        scatter_wait(b)


_LBLK = 4


def _finish_body(x_ref, o_ref):
    for l in range(_LBLK):
        for g in range(G):
            xg = x_ref[l, g * 512 : (g + 1) * 512, :]
            xt = xg.T
            o_ref[l, g * D : (g + 1) * D, :] = jnp.maximum(
                jnp.concatenate([xt[:D, :], xt[D:, :]], axis=1), 0.0
            )


def _finisher(out_lin):
    x = out_lin.reshape(L, (B * G * D) // 128, 128)
    z = pl.pallas_call(
        _finish_body,
        grid=(L // _LBLK,),
        in_specs=[
            pl.BlockSpec((_LBLK, (B * G * D) // 128, 128), lambda i: (i, 0, 0))
        ],
        out_specs=pl.BlockSpec((_LBLK, G * D, B), lambda i: (i, 0, 0)),
        out_shape=jax.ShapeDtypeStruct((L, G * D, B), jnp.float32),
    )(x)
    return jnp.transpose(z, (0, 2, 1))


def kernel(indices, table):
    idx_flat = jnp.transpose(indices.astype(jnp.int32), (2, 1, 0)).reshape(-1)
    out = _gather_scatter(idx_flat, table.astype(jnp.float32))
    return _finisher(out)


# speedup vs baseline: 2.0688x; 1.0230x over previous
import functools

import jax
import jax.numpy as jnp
from jax import lax
from jax.experimental import pallas as pl
from jax.experimental.pallas import tpu as pltpu
from jax.experimental.pallas import tpu_sc as plsc

B, L, G = 1024, 200, 3
D = 64
N = B * L * G
LANES = 16
NW = 32
ROWS_PER_W = N // NW
C = 128
N_CHUNKS = ROWS_PER_W // C
NBUF = 6
K = 3


_mesh = plsc.VectorSubcoreMesh(core_axis_name="c", subcore_axis_name="s")

_scratch = (
    [pltpu.VMEM((C,), jnp.int32) for _ in range(NBUF)]
    + [pltpu.VMEM((C,), jnp.int32) for _ in range(NBUF)]
    + [pltpu.VMEM((C, D), jnp.float32) for _ in range(NBUF)]
    + [pltpu.SemaphoreType.DMA for _ in range(3 * NBUF)]
)


@functools.partial(
    pl.kernel,
    out_type=jax.ShapeDtypeStruct((N, D), jnp.float32),
    mesh=_mesh,
    scratch_types=_scratch,
    compiler_params=pltpu.CompilerParams(use_tc_tiling_on_sc=False),
)
def _gather_scatter(idx_hbm, table_hbm, out_hbm, *scratch):
    ibuf = scratch[0:NBUF]
    sidx = scratch[NBUF : 2 * NBUF]
    rows = scratch[2 * NBUF : 3 * NBUF]
    gsem = scratch[3 * NBUF : 4 * NBUF]
    ssem = scratch[4 * NBUF : 5 * NBUF]
    isem = scratch[5 * NBUF : 6 * NBUF]

    cid = lax.axis_index("c")
    sid = lax.axis_index("s")
    wid = sid * 2 + cid
    wbase = wid * ROWS_PER_W

    def compute_sidx(b, cg):
        c8 = lax.shift_right_logical(cg, 3)
        g = lax.shift_right_logical(c8 * 328, 16)
        l = c8 - L * g
        b0 = lax.shift_left(cg & 7, 7)
        base_s = (
            l * (B * G)
            + g * B
            + lax.shift_left(b0 & 511, 1)
            + lax.shift_right_logical(b0, 9)
        )
        for v in range(C // LANES):
            cv = (lax.iota(jnp.int32, LANES) + (v * LANES)) * 2
            sidx[b][pl.ds(v * LANES, LANES)] = base_s + cv

    def idx_start(c, b):
        base = pl.multiple_of(wbase + c * C, C)
        pltpu.async_copy(idx_hbm.at[pl.ds(base, C)], ibuf[b], isem[b])

    def idx_wait(b):
        pltpu.make_async_copy(idx_hbm.at[pl.ds(0, C)], ibuf[b], isem[b]).wait()

    def gather_start(b):
        pltpu.async_copy(table_hbm.at[ibuf[b]], rows[b], gsem[b])

    def gather_wait(b):
        pltpu.make_async_copy(table_hbm.at[ibuf[b]], rows[b], gsem[b]).wait()

    def scatter_start(b):
        pltpu.async_copy(rows[b], out_hbm.at[sidx[b]], ssem[b])

    def scatter_wait(b):
        pltpu.make_async_copy(rows[b], out_hbm.at[sidx[b]], ssem[b]).wait()

    wchunk = wid * N_CHUNKS

    for c in range(NBUF):
        idx_start(c, c)
    for c in range(NBUF):
        b = c
        if c >= K:
            b2 = c - K
            gather_wait(b2)
            scatter_start(b2)
            idx_start(c + K, b2)
        compute_sidx(b, wchunk + c)
        idx_wait(b)
        gather_start(b)

    @pl.loop(0, (N_CHUNKS - 2 * NBUF) // NBUF)
    def _block(j):
        for b in range(NBUF):
            c = NBUF + j * NBUF + b
            b2 = (b + NBUF - K) % NBUF
            gather_wait(b2)
            scatter_start(b2)
            idx_start(c + K, b2)
            scatter_wait(b)
            compute_sidx(b, wchunk + c)
            idx_wait(b)
            gather_start(b)

    for c in range(N_CHUNKS - NBUF, N_CHUNKS):
        b = c % NBUF
        b2 = (b + NBUF - K) % NBUF
        gather_wait(b2)
        scatter_start(b2)
        if c + K < N_CHUNKS:
            idx_start(c + K, b2)
        scatter_wait(b)
        compute_sidx(b, wchunk + c)
        idx_wait(b)
        gather_start(b)

    for c in range(N_CHUNKS - K, N_CHUNKS):
        b = c % NBUF
        gather_wait(b)
        scatter_start(b)
    for b in range(NBUF):
        scatter_wait(b)


_LBLK = 8


def _finish_body(x_ref, o_ref):
    for l in range(_LBLK):
        for g in range(G):
            xg = x_ref[l, g * 512 : (g + 1) * 512, :]
            xt = xg.T
            o_ref[l, g * D : (g + 1) * D, :] = jnp.maximum(
                jnp.concatenate([xt[:D, :], xt[D:, :]], axis=1), 0.0
            )


def _finisher(out_lin):
    x = out_lin.reshape(L, (B * G * D) // 128, 128)
    z = pl.pallas_call(
        _finish_body,
        grid=(L // _LBLK,),
        in_specs=[
            pl.BlockSpec((_LBLK, (B * G * D) // 128, 128), lambda i: (i, 0, 0))
        ],
        out_specs=pl.BlockSpec((_LBLK, G * D, B), lambda i: (i, 0, 0)),
        out_shape=jax.ShapeDtypeStruct((L, G * D, B), jnp.float32),
    )(x)
    return jnp.transpose(z, (0, 2, 1))


def kernel(indices, table):
    idx_flat = jnp.transpose(indices.astype(jnp.int32), (2, 1, 0)).reshape(-1)
    out = _gather_scatter(idx_flat, table.astype(jnp.float32))
    return _finisher(out)
